# baseline (speedup 1.0000x reference)
import functools

import jax
import jax.numpy as jnp
from jax import lax
from jax.experimental import pallas as pl
from jax.experimental.pallas import tpu as pltpu
from jax.experimental.pallas import tpu_sc as plsc

BT = 256
BTT = 64
FK = 1024
SC_CORES = 2
SC_SUBCORES = 16
NW = SC_CORES * SC_SUBCORES


def _gate_body(x_ref, wg_ref, bg_ref, dest_ref, astart_ref, n256_ref,
               n64_ref):
    T, E = x_ref.shape[0], wg_ref.shape[1]
    x = x_ref[...]
    logits = lax.dot_general(
        x, wg_ref[...].astype(jnp.bfloat16), (((1,), (0,)), ((), ())),
        preferred_element_type=jnp.float32) + bg_ref[...]
    m = jnp.max(logits, axis=1, keepdims=True)
    lane = lax.broadcasted_iota(jnp.int32, (T, E), 1)
    idx = jnp.min(jnp.where(logits == m, lane, E), axis=1, keepdims=True)
    onehot = (lane == idx).astype(jnp.float32)
    counts = jnp.sum(onehot, axis=0, keepdims=True)
    c8 = jnp.floor((counts + 7.0) * 0.125) * 8.0
    sub = lax.broadcasted_iota(jnp.int32, (E, E), 0)
    lan = lax.broadcasted_iota(jnp.int32, (E, E), 1)
    mstrict = (sub < lan).astype(jnp.float32)
    astart_all = lax.dot_general(
        jnp.broadcast_to(c8, (E, E)), mstrict, (((1,), (0,)), ((), ())),
        precision=lax.Precision.HIGHEST, preferred_element_type=jnp.float32)
    astart_row = astart_all[0:1, :]
    r_io = lax.broadcasted_iota(jnp.int32, (T, T), 0)
    c_io = lax.broadcasted_iota(jnp.int32, (T, T), 1)
    lts = (c_io < r_io).astype(jnp.bfloat16)
    ranks_all = lax.dot_general(
        lts, onehot.astype(jnp.bfloat16), (((1,), (0,)), ((), ())),
        preferred_element_type=jnp.float32)
    rank = jnp.sum(ranks_all * onehot, axis=1, keepdims=True)
    astart_sel = jnp.sum(astart_row * onehot, axis=1, keepdims=True)
    dest_ref[...] = (astart_sel + rank).astype(jnp.int32)
    astart_ref[...] = astart_row.astype(jnp.int32)
    n256 = jnp.floor(counts * (1.0 / BT))
    rem = counts - BT * n256
    n256_ref[...] = n256.astype(jnp.int32)
    n64_ref[...] = jnp.floor((rem + (BTT - 1)) * (1.0 / BTT)).astype(jnp.int32)


def _gate_call(x, Wg, bg):
    T, _ = x.shape
    E = Wg.shape[1]
    return pl.pallas_call(
        _gate_body,
        out_shape=(
            jax.ShapeDtypeStruct((T, 1), jnp.int32),
            jax.ShapeDtypeStruct((1, E), jnp.int32),
            jax.ShapeDtypeStruct((1, E), jnp.int32),
            jax.ShapeDtypeStruct((1, E), jnp.int32),
        ),
    )(x, Wg, bg.reshape(1, E))


def _mlp_body(astart_sm, n256_sm, n64_sm, xs_ref, w1_hbm, w2_hbm, b1_ref,
              b2_ref, ys_ref, w1b, w2b, sems):
    E, FT = pl.num_programs(0), pl.num_programs(1)
    e = pl.program_id(0)
    fk = pl.program_id(1)
    t = e * FT + fk
    slot = lax.rem(t, 2)
    nslot = 1 - slot

    def start_copies(step, buf_slot):
        en = step // FT
        fkn = lax.rem(step, FT)
        pltpu.make_async_copy(
            w1_hbm.at[en, :, pl.ds(fkn * FK, FK)], w1b.at[buf_slot],
            sems.at[buf_slot, 0]).start()
        pltpu.make_async_copy(
            w2_hbm.at[en, pl.ds(fkn * FK, FK), :], w2b.at[buf_slot],
            sems.at[buf_slot, 1]).start()

    def wait_copies(step, buf_slot):
        en = step // FT
        fkn = lax.rem(step, FT)
        pltpu.make_async_copy(
            w1_hbm.at[en, :, pl.ds(fkn * FK, FK)], w1b.at[buf_slot],
            sems.at[buf_slot, 0]).wait()
        pltpu.make_async_copy(
            w2_hbm.at[en, pl.ds(fkn * FK, FK), :], w2b.at[buf_slot],
            sems.at[buf_slot, 1]).wait()

    @pl.when(t == 0)
    def _():
        start_copies(0, 0)

    @pl.when(t < E * FT - 1)
    def _():
        start_copies(t + 1, nslot)

    wait_copies(t, slot)

    a0 = astart_sm[e]
    n256 = n256_sm[e]
    n64 = n64_sm[e]
    w1 = w1b[slot].astype(jnp.bfloat16)
    w2 = w2b[slot].astype(jnp.bfloat16)
    b1 = b1_ref[0, 0]
    b2 = b2_ref[0, 0]

    def make_chunk(bt):
        def chunk_at(r0):
            xc = xs_ref[pl.ds(r0, bt), :]
            h = jnp.dot(xc, w1, preferred_element_type=jnp.float32)
            h = jnp.maximum(h + b1[None, :], 0.0).astype(jnp.bfloat16)
            yc = jnp.dot(h, w2, preferred_element_type=jnp.float32)

            @pl.when(fk == 0)
            def _():
                ys_ref[pl.ds(r0, bt), :] = yc + b2[None, :]

            @pl.when(fk != 0)
            def _():
                ys_ref[pl.ds(r0, bt), :] = ys_ref[pl.ds(r0, bt), :] + yc

        return chunk_at

    main_chunk = make_chunk(BT)
    tail_chunk = make_chunk(BTT)

    def main_body(j, carry):
        main_chunk(pl.multiple_of(a0 + j * BT, 8))
        return carry

    lax.fori_loop(0, n256, main_body, 0)
    t0 = a0 + n256 * BT

    def tail_body(j, carry):
        tail_chunk(pl.multiple_of(t0 + j * BTT, 8))
        return carry

    lax.fori_loop(0, n64, tail_body, 0)


def _mlp_call(astart, n256, n64, xs, W1, b1, W2, b2):
    E, D, F = W1.shape
    S = xs.shape[0]
    grid = (E, F // FK)
    grid_spec = pltpu.PrefetchScalarGridSpec(
        num_scalar_prefetch=3,
        grid=grid,
        in_specs=[
            pl.BlockSpec((S, D), lambda e, fk, *_: (0, 0)),
            pl.BlockSpec(memory_space=pltpu.MemorySpace.HBM),
            pl.BlockSpec(memory_space=pltpu.MemorySpace.HBM),
            pl.BlockSpec((1, 1, FK), lambda e, fk, *_: (e, 0, fk)),
            pl.BlockSpec((1, 1, D), lambda e, fk, *_: (e, 0, 0)),
        ],
        out_specs=pl.BlockSpec((S, D), lambda e, fk, *_: (0, 0)),
        scratch_shapes=[
            pltpu.VMEM((2, D, FK), jnp.float32),
            pltpu.VMEM((2, FK, D), jnp.float32),
            pltpu.SemaphoreType.DMA((2, 2)),
        ],
    )
    return pl.pallas_call(
        _mlp_body,
        grid_spec=grid_spec,
        out_shape=jax.ShapeDtypeStruct((S, D), jnp.float32),
        name="mlp",
        compiler_params=pltpu.CompilerParams(
            dimension_semantics=("arbitrary", "arbitrary")),
    )(astart, n256, n64, xs, W1, W2, b1.reshape(E, 1, F),
      b2.reshape(E, 1, D))


def _sc_scatter_call(x, dest, S):
    T, DW = x.shape
    bpw = T // NW
    mesh = plsc.VectorSubcoreMesh(core_axis_name="c", subcore_axis_name="s")

    @functools.partial(
        pl.kernel, mesh=mesh,
        out_type=jax.ShapeDtypeStruct((S, DW), jnp.int32),
        scratch_types=[
            pltpu.VMEM((bpw,), jnp.int32),
            pltpu.VMEM((bpw, DW), jnp.int32),
            pltpu.SemaphoreType.DMA,
        ])
    def scatter_kernel(x_hbm, dest_hbm, xs_hbm, idx_v, rows_v, sem):
        wid = lax.axis_index("s") * SC_CORES + lax.axis_index("c")
        base = wid * bpw
        pltpu.sync_copy(dest_hbm.at[pl.ds(base, bpw)], idx_v)
        pltpu.sync_copy(x_hbm.at[pl.ds(base, bpw)], rows_v)
        pltpu.async_copy(rows_v, xs_hbm.at[idx_v], sem).wait()

    return scatter_kernel(x, dest)


def _sc_gather_call(ys, dest, T):
    S, D = ys.shape
    bpw = T // NW
    mesh = plsc.VectorSubcoreMesh(core_axis_name="c", subcore_axis_name="s")

    @functools.partial(
        pl.kernel, mesh=mesh,
        out_type=jax.ShapeDtypeStruct((T, D), jnp.float32),
        scratch_types=[
            pltpu.VMEM((bpw,), jnp.int32),
            pltpu.VMEM((bpw, D), jnp.float32),
            pltpu.SemaphoreType.DMA,
        ])
    def gather_kernel(ys_hbm, dest_hbm, out_hbm, idx_v, rows_v, sem):
        wid = lax.axis_index("s") * SC_CORES + lax.axis_index("c")
        base = wid * bpw
        pltpu.sync_copy(dest_hbm.at[pl.ds(base, bpw)], idx_v)
        pltpu.async_copy(ys_hbm.at[idx_v], rows_v, sem).wait()
        pltpu.sync_copy(rows_v, out_hbm.at[pl.ds(base, bpw)])

    return gather_kernel(ys, dest)


def kernel(x, Wg, bg, W1, b1, W2, b2):
    T, D = x.shape
    E = Wg.shape[1]
    S = (T + (E - 1) * 7 + BTT - 1 + 7) // 8 * 8
    xb = x.astype(jnp.bfloat16)
    dest2d, astart, n256, n64 = _gate_call(xb, Wg, bg)
    dest = dest2d.reshape(T)
    xw = lax.bitcast_convert_type(xb.reshape(T, D // 2, 2), jnp.int32)
    xsw = _sc_scatter_call(xw, dest, S)
    xs = lax.bitcast_convert_type(xsw, jnp.bfloat16).reshape(S, D)
    ys = _mlp_call(astart.reshape(E), n256.reshape(E), n64.reshape(E),
                   xs, W1, b1, W2, b2)
    return _sc_gather_call(ys, dest, T)


# speedup vs baseline: 1.5907x; 1.5907x over previous
import functools

import jax
import jax.numpy as jnp
from jax import lax
from jax.experimental import pallas as pl
from jax.experimental.pallas import tpu as pltpu
from jax.experimental.pallas import tpu_sc as plsc

BT = 256
BTT = 64
FK = 1024
SC_CORES = 2
SC_SUBCORES = 16
NW = SC_CORES * SC_SUBCORES


def _gate_body(x_ref, wg_ref, bg_ref, dest_ref, astart_ref, n256_ref,
               n64_ref):
    T, E = x_ref.shape[0], wg_ref.shape[1]
    x = x_ref[...].astype(jnp.bfloat16)
    logits = lax.dot_general(
        x, wg_ref[...].astype(jnp.bfloat16), (((1,), (0,)), ((), ())),
        preferred_element_type=jnp.float32) + bg_ref[...]
    m = jnp.max(logits, axis=1, keepdims=True)
    lane = lax.broadcasted_iota(jnp.int32, (T, E), 1)
    idx = jnp.min(jnp.where(logits == m, lane, E), axis=1, keepdims=True)
    onehot = (lane == idx).astype(jnp.float32)
    counts = jnp.sum(onehot, axis=0, keepdims=True)
    c8 = jnp.floor((counts + 7.0) * 0.125) * 8.0
    sub = lax.broadcasted_iota(jnp.int32, (E, E), 0)
    lan = lax.broadcasted_iota(jnp.int32, (E, E), 1)
    mstrict = (sub < lan).astype(jnp.float32)
    astart_all = lax.dot_general(
        jnp.broadcast_to(c8, (E, E)), mstrict, (((1,), (0,)), ((), ())),
        precision=lax.Precision.HIGHEST, preferred_element_type=jnp.float32)
    astart_row = astart_all[0:1, :]
    r_io = lax.broadcasted_iota(jnp.int32, (T, T), 0)
    c_io = lax.broadcasted_iota(jnp.int32, (T, T), 1)
    lts = (c_io < r_io).astype(jnp.bfloat16)
    ranks_all = lax.dot_general(
        lts, onehot.astype(jnp.bfloat16), (((1,), (0,)), ((), ())),
        preferred_element_type=jnp.float32)
    rank = jnp.sum(ranks_all * onehot, axis=1, keepdims=True)
    astart_sel = jnp.sum(astart_row * onehot, axis=1, keepdims=True)
    dest_ref[...] = (astart_sel + rank).astype(jnp.int32)
    astart_ref[...] = astart_row.astype(jnp.int32)
    n256 = jnp.floor(counts * (1.0 / BT))
    rem = counts - BT * n256
    n256_ref[...] = n256.astype(jnp.int32)
    n64_ref[...] = jnp.floor((rem + (BTT - 1)) * (1.0 / BTT)).astype(jnp.int32)


def _gate_call(x, Wg, bg):
    T, _ = x.shape
    E = Wg.shape[1]
    return pl.pallas_call(
        _gate_body,
        out_shape=(
            jax.ShapeDtypeStruct((T, 1), jnp.int32),
            jax.ShapeDtypeStruct((1, E), jnp.int32),
            jax.ShapeDtypeStruct((1, E), jnp.int32),
            jax.ShapeDtypeStruct((1, E), jnp.int32),
        ),
    )(x, Wg, bg.reshape(1, E))


def _mlp_body(astart_sm, n256_sm, n64_sm, xs_ref, w1_hbm, w2_hbm, b1_ref,
              b2_ref, ys_ref, w1b, w2b, sems):
    E, FT = pl.num_programs(0), pl.num_programs(1)
    e = pl.program_id(0)
    fk = pl.program_id(1)
    t = e * FT + fk
    slot = lax.rem(t, 2)
    nslot = 1 - slot

    D = w1b.shape[1]
    DH = D // 2

    def copies(step, buf_slot):
        en = step // FT
        fkn = lax.rem(step, FT)
        return [
            pltpu.make_async_copy(
                w1_hbm.at[en, pl.ds(0, DH), pl.ds(fkn * FK, FK)],
                w1b.at[buf_slot, pl.ds(0, DH)], sems.at[buf_slot, 0]),
            pltpu.make_async_copy(
                w1_hbm.at[en, pl.ds(DH, DH), pl.ds(fkn * FK, FK)],
                w1b.at[buf_slot, pl.ds(DH, DH)], sems.at[buf_slot, 1]),
            pltpu.make_async_copy(
                w2_hbm.at[en, pl.ds(fkn * FK, FK // 2), :],
                w2b.at[buf_slot, pl.ds(0, FK // 2)], sems.at[buf_slot, 2]),
            pltpu.make_async_copy(
                w2_hbm.at[en, pl.ds(fkn * FK + FK // 2, FK // 2), :],
                w2b.at[buf_slot, pl.ds(FK // 2, FK // 2)],
                sems.at[buf_slot, 3]),
        ]

    def start_copies(step, buf_slot):
        for c in copies(step, buf_slot):
            c.start()

    def wait_copies(step, buf_slot):
        for c in copies(step, buf_slot):
            c.wait()

    @pl.when(t == 0)
    def _():
        start_copies(0, 0)

    @pl.when(t < E * FT - 1)
    def _():
        start_copies(t + 1, nslot)

    wait_copies(t, slot)

    a0 = astart_sm[e]
    n256 = n256_sm[e]
    n64 = n64_sm[e]
    w1 = w1b[slot].astype(jnp.bfloat16)
    w2 = w2b[slot].astype(jnp.bfloat16)
    b1 = b1_ref[0, 0]
    b2 = b2_ref[0, 0]

    def make_chunk(bt):
        def chunk_at(r0):
            xc = xs_ref[pl.ds(r0, bt), :].astype(jnp.bfloat16)
            h = jnp.dot(xc, w1, preferred_element_type=jnp.float32)
            h = jnp.maximum(h + b1[None, :], 0.0).astype(jnp.bfloat16)
            yc = jnp.dot(h, w2, preferred_element_type=jnp.float32)

            @pl.when(fk == 0)
            def _():
                ys_ref[pl.ds(r0, bt), :] = yc + b2[None, :]

            @pl.when(fk != 0)
            def _():
                ys_ref[pl.ds(r0, bt), :] = ys_ref[pl.ds(r0, bt), :] + yc

        return chunk_at

    main_chunk = make_chunk(BT)
    tail_chunk = make_chunk(BTT)

    def main_body(j, carry):
        main_chunk(pl.multiple_of(a0 + j * BT, 8))
        return carry

    lax.fori_loop(0, n256, main_body, 0)
    t0 = a0 + n256 * BT

    def tail_body(j, carry):
        tail_chunk(pl.multiple_of(t0 + j * BTT, 8))
        return carry

    lax.fori_loop(0, n64, tail_body, 0)


def _mlp_call(astart, n256, n64, xs, W1, b1, W2, b2):
    E, D, F = W1.shape
    S = xs.shape[0]
    grid = (E, F // FK)
    grid_spec = pltpu.PrefetchScalarGridSpec(
        num_scalar_prefetch=3,
        grid=grid,
        in_specs=[
            pl.BlockSpec((S, D), lambda e, fk, *_: (0, 0)),
            pl.BlockSpec(memory_space=pltpu.MemorySpace.HBM),
            pl.BlockSpec(memory_space=pltpu.MemorySpace.HBM),
            pl.BlockSpec((1, 1, FK), lambda e, fk, *_: (e, 0, fk)),
            pl.BlockSpec((1, 1, D), lambda e, fk, *_: (e, 0, 0)),
        ],
        out_specs=pl.BlockSpec((S, D), lambda e, fk, *_: (0, 0)),
        scratch_shapes=[
            pltpu.VMEM((2, D, FK), jnp.float32),
            pltpu.VMEM((2, FK, D), jnp.float32),
            pltpu.SemaphoreType.DMA((2, 4)),
        ],
    )
    return pl.pallas_call(
        _mlp_body,
        grid_spec=grid_spec,
        out_shape=jax.ShapeDtypeStruct((S, D), jnp.float32),
        compiler_params=pltpu.CompilerParams(
            dimension_semantics=("arbitrary", "arbitrary")),
    )(astart, n256, n64, xs, W1, W2, b1.reshape(E, 1, F),
      b2.reshape(E, 1, D))


def _sc_scatter_call(x, dest, S):
    T, D = x.shape
    bpw = T // NW
    mesh = plsc.VectorSubcoreMesh(core_axis_name="c", subcore_axis_name="s")

    @functools.partial(
        pl.kernel, mesh=mesh,
        out_type=jax.ShapeDtypeStruct((S, D), jnp.float32),
        scratch_types=[
            pltpu.VMEM((bpw,), jnp.int32),
            pltpu.VMEM((bpw, D), jnp.float32),
            pltpu.SemaphoreType.DMA,
        ])
    def scatter_kernel(x_hbm, dest_hbm, xs_hbm, idx_v, rows_v, sem):
        wid = lax.axis_index("s") * SC_CORES + lax.axis_index("c")
        base = wid * bpw
        pltpu.sync_copy(dest_hbm.at[pl.ds(base, bpw)], idx_v)
        pltpu.sync_copy(x_hbm.at[pl.ds(base, bpw)], rows_v)
        pltpu.async_copy(rows_v, xs_hbm.at[idx_v], sem).wait()

    return scatter_kernel(x, dest)


def _sc_gather_call(ys, dest, T):
    S, D = ys.shape
    bpw = T // NW
    mesh = plsc.VectorSubcoreMesh(core_axis_name="c", subcore_axis_name="s")

    @functools.partial(
        pl.kernel, mesh=mesh,
        out_type=jax.ShapeDtypeStruct((T, D), jnp.float32),
        scratch_types=[
            pltpu.VMEM((bpw,), jnp.int32),
            pltpu.VMEM((bpw, D), jnp.float32),
            pltpu.SemaphoreType.DMA,
        ])
    def gather_kernel(ys_hbm, dest_hbm, out_hbm, idx_v, rows_v, sem):
        wid = lax.axis_index("s") * SC_CORES + lax.axis_index("c")
        base = wid * bpw
        pltpu.sync_copy(dest_hbm.at[pl.ds(base, bpw)], idx_v)
        pltpu.async_copy(ys_hbm.at[idx_v], rows_v, sem).wait()
        pltpu.sync_copy(rows_v, out_hbm.at[pl.ds(base, bpw)])

    return gather_kernel(ys, dest)


def kernel(x, Wg, bg, W1, b1, W2, b2):
    T, D = x.shape
    E = Wg.shape[1]
    S = (T + (E - 1) * 7 + BTT - 1 + 7) // 8 * 8
    dest2d, astart, n256, n64 = _gate_call(x, Wg, bg)
    dest = dest2d.reshape(T)
    xs = _sc_scatter_call(x, dest, S)
    ys = _mlp_call(astart.reshape(E), n256.reshape(E), n64.reshape(E),
                   xs, W1, b1, W2, b2)
    return _sc_gather_call(ys, dest, T)


# speedup vs baseline: 1.6606x; 1.0439x over previous
import functools

import jax
import jax.numpy as jnp
from jax import lax
from jax.experimental import pallas as pl
from jax.experimental.pallas import tpu as pltpu
from jax.experimental.pallas import tpu_sc as plsc

BT = 256
BTT = 64
FK = 1024
SC_CORES = 2
SC_SUBCORES = 16
NW = SC_CORES * SC_SUBCORES


def _gate_body(x_ref, wg_ref, bg_ref, dest_ref, astart_ref, n256_ref,
               n64_ref):
    T, E = x_ref.shape[0], wg_ref.shape[1]
    x = x_ref[...].astype(jnp.bfloat16)
    logits = lax.dot_general(
        x, wg_ref[...].astype(jnp.bfloat16), (((1,), (0,)), ((), ())),
        preferred_element_type=jnp.float32) + bg_ref[...]
    m = jnp.max(logits, axis=1, keepdims=True)
    lane = lax.broadcasted_iota(jnp.int32, (T, E), 1)
    idx = jnp.min(jnp.where(logits == m, lane, E), axis=1, keepdims=True)
    onehot = (lane == idx).astype(jnp.float32)
    counts = jnp.sum(onehot, axis=0, keepdims=True)
    c8 = jnp.floor((counts + 7.0) * 0.125) * 8.0
    sub = lax.broadcasted_iota(jnp.int32, (E, E), 0)
    lan = lax.broadcasted_iota(jnp.int32, (E, E), 1)
    mstrict = (sub < lan).astype(jnp.float32)
    astart_all = lax.dot_general(
        jnp.broadcast_to(c8, (E, E)), mstrict, (((1,), (0,)), ((), ())),
        precision=lax.Precision.HIGHEST, preferred_element_type=jnp.float32)
    astart_row = astart_all[0:1, :]
    r_io = lax.broadcasted_iota(jnp.int32, (T, T), 0)
    c_io = lax.broadcasted_iota(jnp.int32, (T, T), 1)
    lts = (c_io < r_io).astype(jnp.bfloat16)
    ranks_all = lax.dot_general(
        lts, onehot.astype(jnp.bfloat16), (((1,), (0,)), ((), ())),
        preferred_element_type=jnp.float32)
    rank = jnp.sum(ranks_all * onehot, axis=1, keepdims=True)
    astart_sel = jnp.sum(astart_row * onehot, axis=1, keepdims=True)
    dest_ref[...] = (astart_sel + rank).astype(jnp.int32)
    astart_ref[...] = astart_row.astype(jnp.int32)
    n256_ref[...] = jnp.floor((counts + (BT - 1)) * (1.0 / BT)).astype(
        jnp.int32)
    n64_ref[...] = jnp.zeros_like(counts).astype(jnp.int32)


def _gate_call(x, Wg, bg):
    T, _ = x.shape
    E = Wg.shape[1]
    return pl.pallas_call(
        _gate_body,
        out_shape=(
            jax.ShapeDtypeStruct((T, 1), jnp.int32),
            jax.ShapeDtypeStruct((1, E), jnp.int32),
            jax.ShapeDtypeStruct((1, E), jnp.int32),
            jax.ShapeDtypeStruct((1, E), jnp.int32),
        ),
    )(x, Wg, bg.reshape(1, E))


def _mlp_body(astart_sm, n256_sm, n64_sm, xs_ref, w1_hbm, w2_hbm, b1_ref,
              b2_ref, ys_ref, w1b, w2b, sems):
    E, FT = pl.num_programs(0), pl.num_programs(1)
    e = pl.program_id(0)
    fk = pl.program_id(1)
    t = e * FT + fk
    slot = lax.rem(t, 2)
    nslot = 1 - slot

    D = w1b.shape[1]
    DH = D // 2

    def copies(step, buf_slot):
        en = step // FT
        fkn = lax.rem(step, FT)
        return [
            pltpu.make_async_copy(
                w1_hbm.at[en, pl.ds(0, DH), pl.ds(fkn * FK, FK)],
                w1b.at[buf_slot, pl.ds(0, DH)], sems.at[buf_slot, 0]),
            pltpu.make_async_copy(
                w1_hbm.at[en, pl.ds(DH, DH), pl.ds(fkn * FK, FK)],
                w1b.at[buf_slot, pl.ds(DH, DH)], sems.at[buf_slot, 1]),
            pltpu.make_async_copy(
                w2_hbm.at[en, pl.ds(fkn * FK, FK // 2), :],
                w2b.at[buf_slot, pl.ds(0, FK // 2)], sems.at[buf_slot, 2]),
            pltpu.make_async_copy(
                w2_hbm.at[en, pl.ds(fkn * FK + FK // 2, FK // 2), :],
                w2b.at[buf_slot, pl.ds(FK // 2, FK // 2)],
                sems.at[buf_slot, 3]),
        ]

    def start_copies(step, buf_slot):
        for c in copies(step, buf_slot):
            c.start()

    def wait_copies(step, buf_slot):
        for c in copies(step, buf_slot):
            c.wait()

    @pl.when(t == 0)
    def _():
        start_copies(0, 0)

    @pl.when(t < E * FT - 1)
    def _():
        start_copies(t + 1, nslot)

    wait_copies(t, slot)

    a0 = astart_sm[e]
    n256 = n256_sm[e]
    n64 = n64_sm[e]
    w1 = w1b[slot].astype(jnp.bfloat16)
    w2 = w2b[slot].astype(jnp.bfloat16)
    b1 = b1_ref[0, 0]
    b2 = b2_ref[0, 0]

    def make_chunk(bt):
        def chunk_at(r0):
            xc = xs_ref[pl.ds(r0, bt), :].astype(jnp.bfloat16)
            h = jnp.dot(xc, w1, preferred_element_type=jnp.float32)
            h = jnp.maximum(h + b1[None, :], 0.0).astype(jnp.bfloat16)
            yc = jnp.dot(h, w2, preferred_element_type=jnp.float32)

            @pl.when(fk == 0)
            def _():
                ys_ref[pl.ds(r0, bt), :] = yc + b2[None, :]

            @pl.when(fk != 0)
            def _():
                ys_ref[pl.ds(r0, bt), :] = ys_ref[pl.ds(r0, bt), :] + yc

        return chunk_at

    main_chunk = make_chunk(BT)

    def main_body(j, carry):
        main_chunk(pl.multiple_of(a0 + j * BT, 8))
        return carry

    lax.fori_loop(0, n256, main_body, 0)
    del n64


def _mlp_call(astart, n256, n64, xs, W1, b1, W2, b2):
    E, D, F = W1.shape
    S = xs.shape[0]
    grid = (E, F // FK)
    grid_spec = pltpu.PrefetchScalarGridSpec(
        num_scalar_prefetch=3,
        grid=grid,
        in_specs=[
            pl.BlockSpec((S, D), lambda e, fk, *_: (0, 0)),
            pl.BlockSpec(memory_space=pltpu.MemorySpace.HBM),
            pl.BlockSpec(memory_space=pltpu.MemorySpace.HBM),
            pl.BlockSpec((1, 1, FK), lambda e, fk, *_: (e, 0, fk)),
            pl.BlockSpec((1, 1, D), lambda e, fk, *_: (e, 0, 0)),
        ],
        out_specs=pl.BlockSpec((S, D), lambda e, fk, *_: (0, 0)),
        scratch_shapes=[
            pltpu.VMEM((2, D, FK), jnp.float32),
            pltpu.VMEM((2, FK, D), jnp.float32),
            pltpu.SemaphoreType.DMA((2, 4)),
        ],
    )
    return pl.pallas_call(
        _mlp_body,
        grid_spec=grid_spec,
        out_shape=jax.ShapeDtypeStruct((S, D), jnp.float32),
        compiler_params=pltpu.CompilerParams(
            dimension_semantics=("arbitrary", "arbitrary")),
    )(astart, n256, n64, xs, W1, W2, b1.reshape(E, 1, F),
      b2.reshape(E, 1, D))


def _sc_scatter_call(x, dest, S):
    T, D = x.shape
    bpw = T // NW
    mesh = plsc.VectorSubcoreMesh(core_axis_name="c", subcore_axis_name="s")

    @functools.partial(
        pl.kernel, mesh=mesh,
        out_type=jax.ShapeDtypeStruct((S, D), jnp.float32),
        scratch_types=[
            pltpu.VMEM((bpw,), jnp.int32),
            pltpu.VMEM((bpw, D), jnp.float32),
            pltpu.SemaphoreType.DMA,
        ])
    def scatter_kernel(x_hbm, dest_hbm, xs_hbm, idx_v, rows_v, sem):
        wid = lax.axis_index("s") * SC_CORES + lax.axis_index("c")
        base = wid * bpw
        pltpu.sync_copy(dest_hbm.at[pl.ds(base, bpw)], idx_v)
        pltpu.sync_copy(x_hbm.at[pl.ds(base, bpw)], rows_v)
        pltpu.async_copy(rows_v, xs_hbm.at[idx_v], sem).wait()

    return scatter_kernel(x, dest)


def _sc_gather_call(ys, dest, T):
    S, D = ys.shape
    bpw = T // NW
    mesh = plsc.VectorSubcoreMesh(core_axis_name="c", subcore_axis_name="s")

    @functools.partial(
        pl.kernel, mesh=mesh,
        out_type=jax.ShapeDtypeStruct((T, D), jnp.float32),
        scratch_types=[
            pltpu.VMEM((bpw,), jnp.int32),
            pltpu.VMEM((bpw, D), jnp.float32),
            pltpu.SemaphoreType.DMA,
        ])
    def gather_kernel(ys_hbm, dest_hbm, out_hbm, idx_v, rows_v, sem):
        wid = lax.axis_index("s") * SC_CORES + lax.axis_index("c")
        base = wid * bpw
        pltpu.sync_copy(dest_hbm.at[pl.ds(base, bpw)], idx_v)
        pltpu.async_copy(ys_hbm.at[idx_v], rows_v, sem).wait()
        pltpu.sync_copy(rows_v, out_hbm.at[pl.ds(base, bpw)])

    return gather_kernel(ys, dest)


def kernel(x, Wg, bg, W1, b1, W2, b2):
    T, D = x.shape
    E = Wg.shape[1]
    S = (T + (E - 1) * 7 + BT - 1 + 7) // 8 * 8
    dest2d, astart, n256, n64 = _gate_call(x, Wg, bg)
    dest = dest2d.reshape(T)
    xs = _sc_scatter_call(x, dest, S)
    ys = _mlp_call(astart.reshape(E), n256.reshape(E), n64.reshape(E),
                   xs, W1, b1, W2, b2)
    return _sc_gather_call(ys, dest, T)


# speedup vs baseline: 1.6659x; 1.0032x over previous
import functools

import jax
import jax.numpy as jnp
from jax import lax
from jax.experimental import pallas as pl
from jax.experimental.pallas import tpu as pltpu
from jax.experimental.pallas import tpu_sc as plsc

BT = 256
BTT = 64
FK = 1024
SC_CORES = 2
SC_SUBCORES = 16
NW = SC_CORES * SC_SUBCORES


def _gate_body(x_ref, wg_ref, bg_ref, dest_ref, astart_ref, n256_ref,
               n64_ref):
    T, E = x_ref.shape[0], wg_ref.shape[1]
    x = x_ref[...].astype(jnp.bfloat16)
    logits = lax.dot_general(
        x, wg_ref[...].astype(jnp.bfloat16), (((1,), (0,)), ((), ())),
        preferred_element_type=jnp.float32) + bg_ref[...]
    m = jnp.max(logits, axis=1, keepdims=True)
    lane = lax.broadcasted_iota(jnp.int32, (T, E), 1)
    idx = jnp.min(jnp.where(logits == m, lane, E), axis=1, keepdims=True)
    onehot = (lane == idx).astype(jnp.float32)
    counts = jnp.sum(onehot, axis=0, keepdims=True)
    c8 = jnp.floor((counts + 7.0) * 0.125) * 8.0
    sub = lax.broadcasted_iota(jnp.int32, (E, E), 0)
    lan = lax.broadcasted_iota(jnp.int32, (E, E), 1)
    mstrict = (sub < lan).astype(jnp.float32)
    astart_all = lax.dot_general(
        jnp.broadcast_to(c8, (E, E)), mstrict, (((1,), (0,)), ((), ())),
        precision=lax.Precision.HIGHEST, preferred_element_type=jnp.float32)
    astart_row = astart_all[0:1, :]
    r_io = lax.broadcasted_iota(jnp.int32, (T, T), 0)
    c_io = lax.broadcasted_iota(jnp.int32, (T, T), 1)
    lts = (c_io < r_io).astype(jnp.bfloat16)
    ranks_all = lax.dot_general(
        lts, onehot.astype(jnp.bfloat16), (((1,), (0,)), ((), ())),
        preferred_element_type=jnp.float32)
    rank = jnp.sum(ranks_all * onehot, axis=1, keepdims=True)
    astart_sel = jnp.sum(astart_row * onehot, axis=1, keepdims=True)
    dest_ref[...] = (astart_sel + rank).astype(jnp.int32)
    astart_ref[...] = astart_row.astype(jnp.int32)
    n256_ref[...] = jnp.floor((counts + (BT - 1)) * (1.0 / BT)).astype(
        jnp.int32)
    n64_ref[...] = jnp.zeros_like(counts).astype(jnp.int32)


def _gate_call(x, Wg, bg):
    T, _ = x.shape
    E = Wg.shape[1]
    return pl.pallas_call(
        _gate_body,
        out_shape=(
            jax.ShapeDtypeStruct((T, 1), jnp.int32),
            jax.ShapeDtypeStruct((1, E), jnp.int32),
            jax.ShapeDtypeStruct((1, E), jnp.int32),
            jax.ShapeDtypeStruct((1, E), jnp.int32),
        ),
    )(x, Wg, bg.reshape(1, E))


def _mlp_body(astart_sm, n256_sm, n64_sm, xs_ref, w1_hbm, w2_hbm, b1_ref,
              b2_ref, ys_ref, w1b, w2b, sems):
    E, FT = pl.num_programs(0), pl.num_programs(1)
    e = pl.program_id(0)
    fk = pl.program_id(1)
    t = e * FT + fk
    slot = lax.rem(t, 2)
    nslot = 1 - slot

    D = w1b.shape[1]
    DH = D // 2

    def copies(step, buf_slot):
        en = step // FT
        fkn = lax.rem(step, FT)
        return [
            pltpu.make_async_copy(
                w1_hbm.at[en, pl.ds(0, DH), pl.ds(fkn * FK, FK)],
                w1b.at[buf_slot, pl.ds(0, DH)], sems.at[buf_slot, 0]),
            pltpu.make_async_copy(
                w1_hbm.at[en, pl.ds(DH, DH), pl.ds(fkn * FK, FK)],
                w1b.at[buf_slot, pl.ds(DH, DH)], sems.at[buf_slot, 1]),
            pltpu.make_async_copy(
                w2_hbm.at[en, pl.ds(fkn * FK, FK // 2), :],
                w2b.at[buf_slot, pl.ds(0, FK // 2)], sems.at[buf_slot, 2]),
            pltpu.make_async_copy(
                w2_hbm.at[en, pl.ds(fkn * FK + FK // 2, FK // 2), :],
                w2b.at[buf_slot, pl.ds(FK // 2, FK // 2)],
                sems.at[buf_slot, 3]),
        ]

    def start_copies(step, buf_slot):
        for c in copies(step, buf_slot):
            c.start()

    def wait_copies(step, buf_slot):
        for c in copies(step, buf_slot):
            c.wait()

    @pl.when(t == 0)
    def _():
        start_copies(0, 0)

    @pl.when(t < E * FT - 1)
    def _():
        start_copies(t + 1, nslot)

    wait_copies(t, slot)

    a0 = astart_sm[e]
    n256 = n256_sm[e]
    n64 = n64_sm[e]
    w1 = w1b[slot]
    w2 = w2b[slot]
    b1 = b1_ref[0, 0]
    b2 = b2_ref[0, 0]

    def make_chunk(bt):
        def chunk_at(r0):
            xc = xs_ref[pl.ds(r0, bt), :]
            h = jnp.dot(xc, w1, preferred_element_type=jnp.float32)
            h = jnp.maximum(h + b1[None, :], 0.0)
            yc = jnp.dot(h, w2, preferred_element_type=jnp.float32)

            @pl.when(fk == 0)
            def _():
                ys_ref[pl.ds(r0, bt), :] = yc + b2[None, :]

            @pl.when(fk != 0)
            def _():
                ys_ref[pl.ds(r0, bt), :] = ys_ref[pl.ds(r0, bt), :] + yc

        return chunk_at

    main_chunk = make_chunk(BT)

    def main_body(j, carry):
        main_chunk(pl.multiple_of(a0 + j * BT, 8))
        return carry

    lax.fori_loop(0, n256, main_body, 0)
    del n64


def _mlp_call(astart, n256, n64, xs, W1, b1, W2, b2):
    E, D, F = W1.shape
    S = xs.shape[0]
    grid = (E, F // FK)
    grid_spec = pltpu.PrefetchScalarGridSpec(
        num_scalar_prefetch=3,
        grid=grid,
        in_specs=[
            pl.BlockSpec((S, D), lambda e, fk, *_: (0, 0)),
            pl.BlockSpec(memory_space=pltpu.MemorySpace.HBM),
            pl.BlockSpec(memory_space=pltpu.MemorySpace.HBM),
            pl.BlockSpec((1, 1, FK), lambda e, fk, *_: (e, 0, fk)),
            pl.BlockSpec((1, 1, D), lambda e, fk, *_: (e, 0, 0)),
        ],
        out_specs=pl.BlockSpec((S, D), lambda e, fk, *_: (0, 0)),
        scratch_shapes=[
            pltpu.VMEM((2, D, FK), jnp.float32),
            pltpu.VMEM((2, FK, D), jnp.float32),
            pltpu.SemaphoreType.DMA((2, 4)),
        ],
    )
    return pl.pallas_call(
        _mlp_body,
        grid_spec=grid_spec,
        out_shape=jax.ShapeDtypeStruct((S, D), jnp.float32),
        compiler_params=pltpu.CompilerParams(
            dimension_semantics=("arbitrary", "arbitrary")),
    )(astart, n256, n64, xs, W1, W2, b1.reshape(E, 1, F),
      b2.reshape(E, 1, D))


def _sc_scatter_call(x, dest, S):
    T, D = x.shape
    bpw = T // NW
    mesh = plsc.VectorSubcoreMesh(core_axis_name="c", subcore_axis_name="s")

    @functools.partial(
        pl.kernel, mesh=mesh,
        out_type=jax.ShapeDtypeStruct((S, D), jnp.float32),
        scratch_types=[
            pltpu.VMEM((bpw,), jnp.int32),
            pltpu.VMEM((bpw, D), jnp.float32),
            pltpu.SemaphoreType.DMA,
        ])
    def scatter_kernel(x_hbm, dest_hbm, xs_hbm, idx_v, rows_v, sem):
        wid = lax.axis_index("s") * SC_CORES + lax.axis_index("c")
        base = wid * bpw
        pltpu.sync_copy(dest_hbm.at[pl.ds(base, bpw)], idx_v)
        pltpu.sync_copy(x_hbm.at[pl.ds(base, bpw)], rows_v)
        pltpu.async_copy(rows_v, xs_hbm.at[idx_v], sem).wait()

    return scatter_kernel(x, dest)


def _sc_gather_call(ys, dest, T):
    S, D = ys.shape
    bpw = T // NW
    mesh = plsc.VectorSubcoreMesh(core_axis_name="c", subcore_axis_name="s")

    @functools.partial(
        pl.kernel, mesh=mesh,
        out_type=jax.ShapeDtypeStruct((T, D), jnp.float32),
        scratch_types=[
            pltpu.VMEM((bpw,), jnp.int32),
            pltpu.VMEM((bpw, D), jnp.float32),
            pltpu.SemaphoreType.DMA,
        ])
    def gather_kernel(ys_hbm, dest_hbm, out_hbm, idx_v, rows_v, sem):
        wid = lax.axis_index("s") * SC_CORES + lax.axis_index("c")
        base = wid * bpw
        pltpu.sync_copy(dest_hbm.at[pl.ds(base, bpw)], idx_v)
        pltpu.async_copy(ys_hbm.at[idx_v], rows_v, sem).wait()
        pltpu.sync_copy(rows_v, out_hbm.at[pl.ds(base, bpw)])

    return gather_kernel(ys, dest)


def kernel(x, Wg, bg, W1, b1, W2, b2):
    T, D = x.shape
    E = Wg.shape[1]
    S = (T + (E - 1) * 7 + BT - 1 + 7) // 8 * 8
    dest2d, astart, n256, n64 = _gate_call(x, Wg, bg)
    dest = dest2d.reshape(T)
    xs = _sc_scatter_call(x, dest, S)
    ys = _mlp_call(astart.reshape(E), n256.reshape(E), n64.reshape(E),
                   xs, W1, b1, W2, b2)
    return _sc_gather_call(ys, dest, T)
